# baseline (device time: 23303 ns/iter reference)
import jax
import jax.numpy as jnp
from jax import lax
from jax.experimental import pallas as pl
from jax.experimental.pallas import tpu as pltpu

N_DEV = 4
E_LOCAL = 4
N_TOK = 512
D_IN = 256
D_OUT = 512
CHUNK = N_TOK // N_DEV


def kernel(x, router_W, route_idx, expert_W):
    def body(x_ref, rw_ref, idx_ref, ew_ref, out_ref,
             acc_ref, comm_ref, send_sems, recv_sems):
        my_pos = lax.axis_index("i")
        left = (my_pos - 1) % N_DEV
        right = (my_pos + 1) % N_DEV

        barrier_sem = pltpu.get_barrier_semaphore()
        for nbr in [left, right]:
            pl.semaphore_signal(
                barrier_sem, inc=1,
                device_id=(nbr,), device_id_type=pl.DeviceIdType.MESH,
            )
        pl.semaphore_wait(barrier_sem, 2)

        xv = x_ref[:, :]
        scores = jnp.dot(xv, rw_ref[:, :], preferred_element_type=jnp.float32)
        m = jnp.max(scores, axis=1, keepdims=True)
        p = jnp.exp(scores - m)
        p = p / jnp.sum(p, axis=1, keepdims=True)

        idx0 = idx_ref[:, 0:1]
        idx1 = idx_ref[:, 1:2]
        iota = lax.broadcasted_iota(jnp.int32, (N_TOK, 16), 1)
        g0 = jnp.sum(jnp.where(iota == idx0, p, 0.0), axis=1, keepdims=True)
        g1 = jnp.sum(jnp.where(iota == idx1, p, 0.0), axis=1, keepdims=True)
        gs = g0 + g1
        g0 = g0 / gs
        g1 = g1 / gs

        acc = jnp.zeros((N_TOK, D_OUT), dtype=jnp.float32)
        for e in range(E_LOCAL):
            ge = my_pos * E_LOCAL + e
            gate = (jnp.where(idx0 == ge, g0, 0.0)
                    + jnp.where(idx1 == ge, g1, 0.0))
            acc = acc + jnp.dot(xv * gate, ew_ref[e],
                                preferred_element_type=jnp.float32)
        acc_ref[:, :] = acc

        for h in range(N_DEV - 1):
            send_slot = h % 2
            recv_slot = (h + 1) % 2
            c_send = (my_pos + (N_DEV - 1 - h)) % N_DEV
            rdma = pltpu.make_async_remote_copy(
                src_ref=acc_ref.at[pl.ds(c_send * CHUNK, CHUNK), :],
                dst_ref=comm_ref.at[recv_slot],
                send_sem=send_sems.at[send_slot],
                recv_sem=recv_sems.at[recv_slot],
                device_id=(right,),
                device_id_type=pl.DeviceIdType.MESH,
            )
            rdma.start()
            rdma.wait()

            c_recv = (my_pos + (N_DEV - 2 - h)) % N_DEV
            acc_ref[pl.ds(c_recv * CHUNK, CHUNK), :] = (
                acc_ref[pl.ds(c_recv * CHUNK, CHUNK), :]
                + comm_ref[recv_slot, :, :]
            )

        out_ref[:, :] = acc_ref[pl.ds(my_pos * CHUNK, CHUNK), :]

    return pl.pallas_call(
        body,
        out_shape=jax.ShapeDtypeStruct((CHUNK, D_OUT), jnp.float32),
        in_specs=[
            pl.BlockSpec(memory_space=pltpu.VMEM),
            pl.BlockSpec(memory_space=pltpu.VMEM),
            pl.BlockSpec(memory_space=pltpu.VMEM),
            pl.BlockSpec(memory_space=pltpu.VMEM),
        ],
        out_specs=pl.BlockSpec(memory_space=pltpu.VMEM),
        scratch_shapes=[
            pltpu.VMEM((N_TOK, D_OUT), jnp.float32),
            pltpu.VMEM((2, CHUNK, D_OUT), jnp.float32),
            pltpu.SemaphoreType.DMA((2,)),
            pltpu.SemaphoreType.DMA((2,)),
        ],
        compiler_params=pltpu.CompilerParams(collective_id=0),
    )(x, router_W, route_idx, expert_W)


# device time: 22600 ns/iter; 1.0311x vs baseline; 1.0311x over previous
import jax
import jax.numpy as jnp
from jax import lax
from jax.experimental import pallas as pl
from jax.experimental.pallas import tpu as pltpu

N_DEV = 4
E_LOCAL = 4
N_TOK = 512
D_IN = 256
D_OUT = 512
N_EXP = 16
CHUNK = N_TOK // N_DEV


def kernel(x, router_W, route_idx, expert_W):
    def body(x_ref, rw_ref, idx_ref, ew_ref, out_ref,
             send_buf, comm_ref, send_sems, recv_sems):
        my_pos = lax.axis_index("i")
        left = (my_pos - 1) % N_DEV
        right = (my_pos + 1) % N_DEV

        barrier_sem = pltpu.get_barrier_semaphore()
        for nbr in [left, right]:
            pl.semaphore_signal(
                barrier_sem, inc=1,
                device_id=(nbr,), device_id_type=pl.DeviceIdType.MESH,
            )
        pl.semaphore_wait(barrier_sem, 2)

        rw = rw_ref[:, :]

        def compute_chunk(c):
            rows = pl.ds(c * CHUNK, CHUNK)
            xc = x_ref[rows, :]
            idx0 = idx_ref[rows, 0:1]
            idx1 = idx_ref[rows, 1:2]
            scores = jnp.dot(xc, rw, preferred_element_type=jnp.float32)
            m = jnp.max(scores, axis=1, keepdims=True)
            p = jnp.exp(scores - m)
            p = p / jnp.sum(p, axis=1, keepdims=True)
            iota = lax.broadcasted_iota(jnp.int32, (CHUNK, N_EXP), 1)
            g0 = jnp.sum(jnp.where(iota == idx0, p, 0.0), axis=1, keepdims=True)
            g1 = jnp.sum(jnp.where(iota == idx1, p, 0.0), axis=1, keepdims=True)
            gs = g0 + g1
            g0 = g0 / gs
            g1 = g1 / gs
            acc = jnp.zeros((CHUNK, D_OUT), dtype=jnp.float32)
            for e in range(E_LOCAL):
                ge = my_pos * E_LOCAL + e
                gate = (jnp.where(idx0 == ge, g0, 0.0)
                        + jnp.where(idx1 == ge, g1, 0.0))
                acc = acc + jnp.dot(xc * gate, ew_ref[e],
                                    preferred_element_type=jnp.float32)
            return acc

        def make_rdma(h):
            slot = h % 2
            return pltpu.make_async_remote_copy(
                src_ref=send_buf.at[slot],
                dst_ref=comm_ref.at[slot],
                send_sem=send_sems.at[slot],
                recv_sem=recv_sems.at[slot],
                device_id=(right,),
                device_id_type=pl.DeviceIdType.MESH,
            )

        send_buf[0] = compute_chunk((my_pos + 3) % N_DEV)
        rdmas = [make_rdma(0)]
        rdmas[0].start()

        for h in range(1, N_DEV - 1):
            val = compute_chunk((my_pos + 3 - h) % N_DEV)
            if h >= 2:
                rdmas[h - 2].wait_send()
            rdmas[h - 1].wait_recv()
            send_buf[h % 2] = val + comm_ref[(h - 1) % 2]
            rdmas.append(make_rdma(h))
            rdmas[h].start()

        val = compute_chunk(my_pos)
        rdmas[N_DEV - 2].wait_recv()
        out_ref[:, :] = val + comm_ref[(N_DEV - 2) % 2]

        rdmas[N_DEV - 3].wait_send()
        rdmas[N_DEV - 2].wait_send()

    return pl.pallas_call(
        body,
        out_shape=jax.ShapeDtypeStruct((CHUNK, D_OUT), jnp.float32),
        in_specs=[
            pl.BlockSpec(memory_space=pltpu.VMEM),
            pl.BlockSpec(memory_space=pltpu.VMEM),
            pl.BlockSpec(memory_space=pltpu.VMEM),
            pl.BlockSpec(memory_space=pltpu.VMEM),
        ],
        out_specs=pl.BlockSpec(memory_space=pltpu.VMEM),
        scratch_shapes=[
            pltpu.VMEM((2, CHUNK, D_OUT), jnp.float32),
            pltpu.VMEM((2, CHUNK, D_OUT), jnp.float32),
            pltpu.SemaphoreType.DMA((2,)),
            pltpu.SemaphoreType.DMA((2,)),
        ],
        compiler_params=pltpu.CompilerParams(collective_id=0),
    )(x, router_W, route_idx, expert_W)


# device time: 18425 ns/iter; 1.2647x vs baseline; 1.2266x over previous
import jax
import jax.numpy as jnp
from jax import lax
from jax.experimental import pallas as pl
from jax.experimental.pallas import tpu as pltpu

N_DEV = 4
E_LOCAL = 4
N_TOK = 512
D_IN = 256
D_OUT = 512
N_EXP = 16
CHUNK = N_TOK // N_DEV


def kernel(x, router_W, route_idx, expert_W):
    def body(x_ref, rw_ref, idx_ref, ew_ref, out_ref,
             send_buf, comm_ref, send_sems, recv_sems):
        my_pos = lax.axis_index("i")
        left = (my_pos - 1) % N_DEV
        right = (my_pos + 1) % N_DEV

        barrier_sem = pltpu.get_barrier_semaphore()
        for nbr in [left, right]:
            pl.semaphore_signal(
                barrier_sem, inc=1,
                device_id=(nbr,), device_id_type=pl.DeviceIdType.MESH,
            )
        pl.semaphore_wait(barrier_sem, 2)

        rw = rw_ref[:, :]

        def compute_chunk(c):
            rows = pl.ds(c * CHUNK, CHUNK)
            xc = x_ref[rows, :]
            idx0 = idx_ref[rows, 0:1]
            idx1 = idx_ref[rows, 1:2]
            scores = jnp.dot(xc, rw, preferred_element_type=jnp.float32)
            m = jnp.max(scores, axis=1, keepdims=True)
            p = jnp.exp(scores - m)
            p = p / jnp.sum(p, axis=1, keepdims=True)
            iota = lax.broadcasted_iota(jnp.int32, (CHUNK, N_EXP), 1)
            g0 = jnp.sum(jnp.where(iota == idx0, p, 0.0), axis=1, keepdims=True)
            g1 = jnp.sum(jnp.where(iota == idx1, p, 0.0), axis=1, keepdims=True)
            gs = g0 + g1
            g0 = g0 / gs
            g1 = g1 / gs
            acc = jnp.zeros((CHUNK, D_OUT), dtype=jnp.float32)
            for e in range(E_LOCAL):
                ge = my_pos * E_LOCAL + e
                gate = (jnp.where(idx0 == ge, g0, 0.0)
                        + jnp.where(idx1 == ge, g1, 0.0))
                acc = acc + jnp.dot(xc * gate, ew_ref[e],
                                    preferred_element_type=jnp.float32)
            return acc

        def make_rdma(h):
            slot = h % 2
            return pltpu.make_async_remote_copy(
                src_ref=send_buf.at[slot],
                dst_ref=comm_ref.at[slot],
                send_sem=send_sems.at[slot],
                recv_sem=recv_sems.at[slot],
                device_id=(right,),
                device_id_type=pl.DeviceIdType.MESH,
            )

        send_buf[0] = compute_chunk((my_pos + 3) % N_DEV).astype(jnp.bfloat16)
        rdmas = [make_rdma(0)]
        rdmas[0].start()

        for h in range(1, N_DEV - 1):
            val = compute_chunk((my_pos + 3 - h) % N_DEV)
            if h >= 2:
                rdmas[h - 2].wait_send()
            rdmas[h - 1].wait_recv()
            send_buf[h % 2] = (
                val + comm_ref[(h - 1) % 2].astype(jnp.float32)
            ).astype(jnp.bfloat16)
            rdmas.append(make_rdma(h))
            rdmas[h].start()

        val = compute_chunk(my_pos)
        rdmas[N_DEV - 2].wait_recv()
        out_ref[:, :] = val + comm_ref[(N_DEV - 2) % 2].astype(jnp.float32)

        rdmas[N_DEV - 3].wait_send()
        rdmas[N_DEV - 2].wait_send()

    return pl.pallas_call(
        body,
        out_shape=jax.ShapeDtypeStruct((CHUNK, D_OUT), jnp.float32),
        in_specs=[
            pl.BlockSpec(memory_space=pltpu.VMEM),
            pl.BlockSpec(memory_space=pltpu.VMEM),
            pl.BlockSpec(memory_space=pltpu.VMEM),
            pl.BlockSpec(memory_space=pltpu.VMEM),
        ],
        out_specs=pl.BlockSpec(memory_space=pltpu.VMEM),
        scratch_shapes=[
            pltpu.VMEM((2, CHUNK, D_OUT), jnp.bfloat16),
            pltpu.VMEM((2, CHUNK, D_OUT), jnp.bfloat16),
            pltpu.SemaphoreType.DMA((2,)),
            pltpu.SemaphoreType.DMA((2,)),
        ],
        compiler_params=pltpu.CompilerParams(collective_id=0),
    )(x, router_W, route_idx, expert_W)


# device time: 14102 ns/iter; 1.6525x vs baseline; 1.3066x over previous
import jax
import jax.numpy as jnp
from jax import lax
from jax.experimental import pallas as pl
from jax.experimental.pallas import tpu as pltpu

N_DEV = 4
E_LOCAL = 4
N_TOK = 512
D_IN = 256
D_OUT = 512
N_EXP = 16
CHUNK = N_TOK // N_DEV

_OFFSETS = (2, 1, 3)


def kernel(x, router_W, route_idx, expert_W):
    def body(x_ref, rw_ref, idx_ref, ew_ref, out_ref,
             send_buf, comm_ref, send_sems, recv_sems):
        my_pos = lax.axis_index("i")

        barrier_sem = pltpu.get_barrier_semaphore()
        for d in _OFFSETS:
            pl.semaphore_signal(
                barrier_sem, inc=1,
                device_id=((my_pos + d) % N_DEV,),
                device_id_type=pl.DeviceIdType.MESH,
            )
        pl.semaphore_wait(barrier_sem, len(_OFFSETS))

        rw = rw_ref[:, :]

        def compute_chunk(c):
            rows = pl.ds(c * CHUNK, CHUNK)
            xc = x_ref[rows, :]
            idx0 = idx_ref[rows, 0:1]
            idx1 = idx_ref[rows, 1:2]
            scores = jnp.dot(xc, rw, preferred_element_type=jnp.float32)
            m = jnp.max(scores, axis=1, keepdims=True)
            p = jnp.exp(scores - m)
            p = p / jnp.sum(p, axis=1, keepdims=True)
            iota = lax.broadcasted_iota(jnp.int32, (CHUNK, N_EXP), 1)
            g0 = jnp.sum(jnp.where(iota == idx0, p, 0.0), axis=1, keepdims=True)
            g1 = jnp.sum(jnp.where(iota == idx1, p, 0.0), axis=1, keepdims=True)
            gs = g0 + g1
            g0 = g0 / gs
            g1 = g1 / gs
            acc = jnp.zeros((CHUNK, D_OUT), dtype=jnp.float32)
            for e in range(E_LOCAL):
                ge = my_pos * E_LOCAL + e
                gate = (jnp.where(idx0 == ge, g0, 0.0)
                        + jnp.where(idx1 == ge, g1, 0.0))
                acc = acc + jnp.dot(xc * gate, ew_ref[e],
                                    preferred_element_type=jnp.float32)
            return acc

        rdmas = []
        for k, d in enumerate(_OFFSETS):
            target = (my_pos + d) % N_DEV
            send_buf[k] = compute_chunk(target).astype(jnp.bfloat16)
            rdma = pltpu.make_async_remote_copy(
                src_ref=send_buf.at[k],
                dst_ref=comm_ref.at[k],
                send_sem=send_sems.at[k],
                recv_sem=recv_sems.at[k],
                device_id=(target,),
                device_id_type=pl.DeviceIdType.MESH,
            )
            rdma.start()
            rdmas.append(rdma)

        acc = compute_chunk(my_pos)

        for rdma in rdmas:
            rdma.wait_recv()
        out_ref[:, :] = (
            acc
            + comm_ref[0].astype(jnp.float32)
            + comm_ref[1].astype(jnp.float32)
            + comm_ref[2].astype(jnp.float32)
        )

        for rdma in rdmas:
            rdma.wait_send()

    return pl.pallas_call(
        body,
        out_shape=jax.ShapeDtypeStruct((CHUNK, D_OUT), jnp.float32),
        in_specs=[
            pl.BlockSpec(memory_space=pltpu.VMEM),
            pl.BlockSpec(memory_space=pltpu.VMEM),
            pl.BlockSpec(memory_space=pltpu.VMEM),
            pl.BlockSpec(memory_space=pltpu.VMEM),
        ],
        out_specs=pl.BlockSpec(memory_space=pltpu.VMEM),
        scratch_shapes=[
            pltpu.VMEM((3, CHUNK, D_OUT), jnp.bfloat16),
            pltpu.VMEM((3, CHUNK, D_OUT), jnp.bfloat16),
            pltpu.SemaphoreType.DMA((3,)),
            pltpu.SemaphoreType.DMA((3,)),
        ],
        compiler_params=pltpu.CompilerParams(collective_id=0),
    )(x, router_W, route_idx, expert_W)
